# baseline (speedup 1.0000x reference)
import jax
import jax.numpy as jnp
from jax import lax
from jax.experimental import pallas as pl
from jax.experimental.pallas import tpu as pltpu
from jax.experimental.pallas import tpu_sc as plsc

_D = 64
_C = 64
_NBUF = 4
_NC = 2
_NS = 16
_NW = _NC * _NS


def _gather_rows(pe, idx2d, n):
    mesh = plsc.VectorSubcoreMesh(core_axis_name="c", subcore_axis_name="s")
    per_w = n // _NW
    iters = per_w // _C

    @pl.kernel(
        out_type=jax.ShapeDtypeStruct((n, _D), pe.dtype),
        mesh=mesh,
        scratch_types=[
            pltpu.VMEM((per_w,), jnp.int32),
            pltpu.VMEM((_NBUF, _C, _D), jnp.float32),
            pltpu.SemaphoreType.DMA,
            pltpu.SemaphoreType.DMA,
            pltpu.SemaphoreType.DMA,
        ],
        compiler_params=pltpu.CompilerParams(use_tc_tiling_on_sc=False),
    )
    def gather_kernel(pe_hbm, idx_hbm, out_hbm, idx_v, rows_v, sem_i, sem_g,
                      sem_o):
        cid = lax.axis_index("c")
        sid = lax.axis_index("s")
        wid = sid * _NC + cid
        base = wid * per_w
        pltpu.async_copy(idx_hbm.at[0, pl.ds(base, per_w)], idx_v, sem_i).wait()

        def fire_gather(i, b):
            pltpu.make_async_copy(
                pe_hbm.at[idx_v.at[pl.ds(i * _C, _C)]], rows_v.at[b], sem_g
            ).start()

        def wait_gather(b):
            pltpu.make_async_copy(
                pe_hbm.at[idx_v.at[pl.ds(0, _C)]], rows_v.at[b], sem_g
            ).wait()

        def fire_out(i, b):
            pltpu.make_async_copy(
                rows_v.at[b], out_hbm.at[pl.ds(base + i * _C, _C)], sem_o
            ).start()

        def wait_out_one():
            pltpu.make_async_copy(
                rows_v.at[0], out_hbm.at[pl.ds(base, _C)], sem_o
            ).wait()

        for b in range(_NBUF - 1):
            fire_gather(b, b)

        @pl.loop(0, iters, step=_NBUF)
        def _(i0):
            for b in range(_NBUF):
                i = i0 + b
                wait_gather(b)
                fire_out(i, b)
                ni = i + _NBUF - 1
                nb = (b + _NBUF - 1) % _NBUF

                @pl.when(jnp.logical_and(ni < iters, i > 0))
                def _():
                    wait_out_one()
                    fire_gather(ni, nb)

                @pl.when(jnp.logical_and(ni < iters, i == 0))
                def _():
                    fire_gather(ni, nb)

        for _unused in range(_NBUF):
            wait_out_one()

    return gather_kernel(pe, idx2d)


def kernel(x, pe):
    b, t = x.shape
    n = b * t
    idx2d = x.reshape(1, n).astype(jnp.int32)
    out = _gather_rows(pe, idx2d, n)
    return out.reshape(b, t, _D)


# speedup vs baseline: 1.9293x; 1.9293x over previous
import jax
import jax.numpy as jnp
from jax.experimental import pallas as pl
from jax.experimental.pallas import tpu as pltpu
from jax.experimental.pallas import tpu_sc as plsc

_D = 64
_W = 64


def _gather_rows(pe, idx2d, n):
    mesh = plsc.VectorSubcoreMesh(core_axis_name="c", subcore_axis_name="s")

    @pl.kernel(
        out_type=jax.ShapeDtypeStruct((n, _D), pe.dtype),
        mesh=mesh,
        compiler_params=pltpu.CompilerParams(use_tc_tiling_on_sc=False),
    )
    def gather_kernel(pe_hbm, idx_hbm, out_hbm):
        def body(idx_vmem, out_vmem):
            pltpu.sync_copy(pe_hbm.at[idx_vmem.at[0]], out_vmem)

        pltpu.emit_pipeline(
            body,
            grid=(n // _W,),
            in_specs=[pl.BlockSpec((1, _W), index_map=lambda i: (0, i))],
            out_specs=[pl.BlockSpec((_W, _D), index_map=lambda i: (i, 0))],
            core_axis_name=("c", "s"),
            dimension_semantics=(pltpu.PARALLEL,),
        )(idx_hbm, out_hbm)

    return gather_kernel(pe, idx2d)


def kernel(x, pe):
    b, t = x.shape
    n = b * t
    idx2d = x.reshape(1, n).astype(jnp.int32)
    out = _gather_rows(pe, idx2d, n)
    return out.reshape(b, t, _D)


# speedup vs baseline: 1.9468x; 1.0091x over previous
import jax
import jax.numpy as jnp
from jax.experimental import pallas as pl
from jax.experimental.pallas import tpu as pltpu
from jax.experimental.pallas import tpu_sc as plsc

_D = 64
_W = 64


def _gather_rows(pe, idx2d, n):
    mesh = plsc.VectorSubcoreMesh(core_axis_name="c", subcore_axis_name="s")

    @pl.kernel(
        out_type=jax.ShapeDtypeStruct((n, _D), pe.dtype),
        mesh=mesh,
        compiler_params=pltpu.CompilerParams(use_tc_tiling_on_sc=False),
    )
    def gather_kernel(pe_hbm, idx_hbm, out_hbm):
        def body(idx_vmem, out_vmem):
            pltpu.sync_copy(pe_hbm.at[idx_vmem.at[0]], out_vmem)

        nw = 32
        steps = n // (_W * nw)
        pltpu.emit_pipeline(
            body,
            grid=(nw, steps),
            in_specs=[
                pl.BlockSpec((1, _W), index_map=lambda w, j: (0, j * nw + w))
            ],
            out_specs=[
                pl.BlockSpec((_W, _D), index_map=lambda w, j: (j * nw + w, 0))
            ],
            core_axis_name=("c", "s"),
            dimension_semantics=(pltpu.PARALLEL, pltpu.ARBITRARY),
        )(idx_hbm, out_hbm)

    return gather_kernel(pe, idx2d)


def kernel(x, pe):
    b, t = x.shape
    n = b * t
    idx2d = x.reshape(1, n).astype(jnp.int32)
    out = _gather_rows(pe, idx2d, n)
    return out.reshape(b, t, _D)
